# Initial kernel scaffold; baseline (speedup 1.0000x reference)
import jax
import jax.numpy as jnp
from jax.experimental import pallas as pl


def kernel(x, cutoffs, hash_values):
    raise NotImplementedError("write your pallas kernel here")


# speedup vs baseline: 3.3745x; 3.3745x over previous
import functools

import jax
import jax.numpy as jnp
from jax import lax
from jax.experimental import pallas as pl
from jax.experimental.pallas import tpu as pltpu
from jax.experimental.pallas import tpu_sc as plsc

_NW = 32
_L = 16


def _wid():
    return lax.axis_index("s") * 2 + lax.axis_index("c")


@functools.lru_cache(maxsize=None)
def _make_reduce(n, chunk):
    per_w = n // _NW
    nch = per_w // chunk
    mesh = plsc.VectorSubcoreMesh(core_axis_name="c", subcore_axis_name="s")

    @functools.partial(
        pl.kernel,
        mesh=mesh,
        out_type=jax.ShapeDtypeStruct((2, _NW, _L), jnp.float32),
        scratch_types=[
            pltpu.VMEM((chunk,), jnp.float32),
            pltpu.VMEM((chunk,), jnp.float32),
            pltpu.VMEM((_L,), jnp.float32),
            pltpu.VMEM((_L,), jnp.float32),
            pltpu.SemaphoreType.DMA,
            pltpu.SemaphoreType.DMA,
        ],
    )
    def reduce_k(x_hbm, out_hbm, buf0, buf1, sv, qv, sem0, sem1):
        wid = _wid()
        base = wid * per_w
        bufs = (buf0, buf1)
        sems = (sem0, sem1)
        copies = [None, None]
        copies[0] = pltpu.async_copy(x_hbm.at[pl.ds(base, chunk)], buf0, sem0)
        acc_s = jnp.zeros((_L,), jnp.float32)
        acc_q = jnp.zeros((_L,), jnp.float32)
        for ch in range(nch):
            b = ch % 2
            if ch + 1 < nch:
                nb = (ch + 1) % 2
                copies[nb] = pltpu.async_copy(
                    x_hbm.at[pl.ds(base + (ch + 1) * chunk, chunk)],
                    bufs[nb], sems[nb])
            copies[b].wait()
            buf = bufs[b]

            def body(i, carry):
                s, q = carry
                v = buf[pl.ds(i * _L, _L)]
                return s + v, q + v * v

            cs, cq = lax.fori_loop(
                0, chunk // _L, body,
                (jnp.zeros((_L,), jnp.float32), jnp.zeros((_L,), jnp.float32)))
            acc_s = acc_s + cs
            acc_q = acc_q + cq
        sv[...] = acc_s
        qv[...] = acc_q
        pltpu.sync_copy(sv, out_hbm.at[0, wid])
        pltpu.sync_copy(qv, out_hbm.at[1, wid])

    return reduce_k


@functools.lru_cache(maxsize=None)
def _make_map(n, chunk, n_thr):
    per_w = n // _NW
    nch = per_w // chunk
    mesh = plsc.VectorSubcoreMesh(core_axis_name="c", subcore_axis_name="s")

    @functools.partial(
        pl.kernel,
        mesh=mesh,
        out_type=jax.ShapeDtypeStruct((n,), jnp.float32),
        scratch_types=[
            pltpu.VMEM((chunk,), jnp.float32),
            pltpu.VMEM((chunk,), jnp.float32),
            pltpu.VMEM((chunk,), jnp.float32),
            pltpu.VMEM((chunk,), jnp.float32),
            pltpu.VMEM((48,), jnp.float32),
            pltpu.SemaphoreType.DMA,
            pltpu.SemaphoreType.DMA,
            pltpu.SemaphoreType.DMA,
            pltpu.SemaphoreType.DMA,
        ],
    )
    def map_k(x_hbm, par_hbm, out_hbm, ib0, ib1, ob0, ob1, pv,
              si0, si1, so0, so1):
        wid = _wid()
        base = wid * per_w
        pltpu.sync_copy(par_hbm, pv)
        tvec = pv[pl.ds(0, _L)]
        dvec = pv[pl.ds(16, _L)]
        hvec = pv[pl.ds(32, _L)]
        ts = [tvec[i] for i in range(n_thr)]
        dl = [dvec[i] for i in range(n_thr)]
        h0 = hvec[0]
        ibufs = (ib0, ib1)
        obufs = (ob0, ob1)
        isems = (si0, si1)
        osems = (so0, so1)
        in_copies = [None, None]
        out_copies = [None, None]
        in_copies[0] = pltpu.async_copy(
            x_hbm.at[pl.ds(base, chunk)], ib0, si0)
        for ch in range(nch):
            b = ch % 2
            if ch + 1 < nch:
                nb = (ch + 1) % 2
                in_copies[nb] = pltpu.async_copy(
                    x_hbm.at[pl.ds(base + (ch + 1) * chunk, chunk)],
                    ibufs[nb], isems[nb])
            in_copies[b].wait()
            if out_copies[b] is not None:
                out_copies[b].wait()
            ib = ibufs[b]
            ob = obufs[b]

            def body(i, c):
                v = ib[pl.ds(i * _L, _L)]
                acc = jnp.full((_L,), h0, jnp.float32)
                for t, d in zip(ts, dl):
                    acc = jnp.where(v > t, acc + d, acc)
                ob[pl.ds(i * _L, _L)] = acc
                return c

            lax.fori_loop(0, chunk // _L, body, 0)
            out_copies[b] = pltpu.async_copy(
                obufs[b], out_hbm.at[pl.ds(base + ch * chunk, chunk)],
                osems[b])
        for oc in out_copies:
            if oc is not None:
                oc.wait()

    return map_k


def kernel(x, cutoffs, hash_values):
    n = x.shape[0]
    nl = hash_values.shape[0]
    x = x.astype(jnp.float32)
    part = _make_reduce(n, 16384)(x)
    s = jnp.sum(part[0])
    q = jnp.sum(part[1])
    mean = s / n
    var = (q - s * s / n) / (n - 1)
    denom = jnp.sqrt(var) + 1e-6
    t = cutoffs[1:-1] * denom + mean
    deltas = hash_values[1:] - hash_values[:-1]
    params = jnp.zeros((48,), jnp.float32)
    params = (params.at[0:nl - 1].set(t)
                    .at[16:16 + nl - 1].set(deltas)
                    .at[32].set(hash_values[0]))
    return _make_map(n, 16384, nl - 1)(x, params)


# speedup vs baseline: 4.5723x; 1.3550x over previous
import functools

import jax
import jax.numpy as jnp
from jax import lax
from jax.experimental import pallas as pl
from jax.experimental.pallas import tpu as pltpu
from jax.experimental.pallas import tpu_sc as plsc

_NW = 32
_L = 16


def _wid():
    return lax.axis_index("s") * 2 + lax.axis_index("c")


@functools.lru_cache(maxsize=None)
def _make_reduce(n, chunk):
    per_w = n // _NW
    nch = per_w // chunk
    mesh = plsc.VectorSubcoreMesh(core_axis_name="c", subcore_axis_name="s")

    @functools.partial(
        pl.kernel,
        mesh=mesh,
        out_type=jax.ShapeDtypeStruct((2, _NW, _L), jnp.float32),
        scratch_types=[
            pltpu.VMEM((chunk,), jnp.float32),
            pltpu.VMEM((chunk,), jnp.float32),
            pltpu.VMEM((_L,), jnp.float32),
            pltpu.VMEM((_L,), jnp.float32),
            pltpu.SemaphoreType.DMA,
            pltpu.SemaphoreType.DMA,
        ],
    )
    def reduce_k(x_hbm, out_hbm, buf0, buf1, sv, qv, sem0, sem1):
        wid = _wid()
        base = wid * per_w
        bufs = (buf0, buf1)
        sems = (sem0, sem1)
        copies = [None, None]
        copies[0] = pltpu.async_copy(x_hbm.at[pl.ds(base, chunk)], buf0, sem0)
        acc_s = jnp.zeros((_L,), jnp.float32)
        acc_q = jnp.zeros((_L,), jnp.float32)
        for ch in range(nch):
            b = ch % 2
            if ch + 1 < nch:
                nb = (ch + 1) % 2
                copies[nb] = pltpu.async_copy(
                    x_hbm.at[pl.ds(base + (ch + 1) * chunk, chunk)],
                    bufs[nb], sems[nb])
            copies[b].wait()
            buf = bufs[b]
            U = 8
            zz = tuple(jnp.zeros((_L,), jnp.float32) for _ in range(U))

            def body(j, carry):
                ss, qq = carry
                ns, nq = [], []
                for u in range(U):
                    v = buf[pl.ds(j * (U * _L) + u * _L, _L)]
                    ns.append(ss[u] + v)
                    nq.append(qq[u] + v * v)
                return tuple(ns), tuple(nq)

            ss, qq = lax.fori_loop(0, chunk // (U * _L), body, (zz, zz))
            for u in range(U):
                acc_s = acc_s + ss[u]
                acc_q = acc_q + qq[u]
        sv[...] = acc_s
        qv[...] = acc_q
        pltpu.sync_copy(sv, out_hbm.at[0, wid])
        pltpu.sync_copy(qv, out_hbm.at[1, wid])

    return reduce_k


@functools.lru_cache(maxsize=None)
def _make_map(n, chunk, n_thr):
    per_w = n // _NW
    nch = per_w // chunk
    mesh = plsc.VectorSubcoreMesh(core_axis_name="c", subcore_axis_name="s")

    @functools.partial(
        pl.kernel,
        mesh=mesh,
        out_type=jax.ShapeDtypeStruct((n,), jnp.float32),
        scratch_types=[
            pltpu.VMEM((chunk,), jnp.float32),
            pltpu.VMEM((chunk,), jnp.float32),
            pltpu.VMEM((chunk,), jnp.float32),
            pltpu.VMEM((chunk,), jnp.float32),
            pltpu.VMEM((48,), jnp.float32),
            pltpu.SemaphoreType.DMA,
            pltpu.SemaphoreType.DMA,
            pltpu.SemaphoreType.DMA,
            pltpu.SemaphoreType.DMA,
        ],
    )
    def map_k(x_hbm, par_hbm, out_hbm, ib0, ib1, ob0, ob1, pv,
              si0, si1, so0, so1):
        wid = _wid()
        base = wid * per_w
        pltpu.sync_copy(par_hbm, pv)
        tvec = pv[pl.ds(0, _L)]
        dvec = pv[pl.ds(16, _L)]
        hvec = pv[pl.ds(32, _L)]
        ts = [tvec[i] for i in range(n_thr)]
        dl = [dvec[i] for i in range(n_thr)]
        h0 = hvec[0]
        ibufs = (ib0, ib1)
        obufs = (ob0, ob1)
        isems = (si0, si1)
        osems = (so0, so1)
        in_copies = [None, None]
        out_copies = [None, None]
        in_copies[0] = pltpu.async_copy(
            x_hbm.at[pl.ds(base, chunk)], ib0, si0)
        for ch in range(nch):
            b = ch % 2
            if ch + 1 < nch:
                nb = (ch + 1) % 2
                in_copies[nb] = pltpu.async_copy(
                    x_hbm.at[pl.ds(base + (ch + 1) * chunk, chunk)],
                    ibufs[nb], isems[nb])
            in_copies[b].wait()
            if out_copies[b] is not None:
                out_copies[b].wait()
            ib = ibufs[b]
            ob = obufs[b]

            @plsc.parallel_loop(0, chunk, _L, unroll=8)
            def _(i):
                v = ib[pl.ds(i, _L)]
                acc = jnp.full((_L,), h0, jnp.float32)
                for t, d in zip(ts, dl):
                    acc = jnp.where(v > t, acc + d, acc)
                ob[pl.ds(i, _L)] = acc
            out_copies[b] = pltpu.async_copy(
                obufs[b], out_hbm.at[pl.ds(base + ch * chunk, chunk)],
                osems[b])
        for oc in out_copies:
            if oc is not None:
                oc.wait()

    return map_k


def kernel(x, cutoffs, hash_values):
    n = x.shape[0]
    nl = hash_values.shape[0]
    x = x.astype(jnp.float32)
    part = _make_reduce(n, 16384)(x)
    s = jnp.sum(part[0])
    q = jnp.sum(part[1])
    mean = s / n
    var = (q - s * s / n) / (n - 1)
    denom = jnp.sqrt(var) + 1e-6
    t = cutoffs[1:-1] * denom + mean
    deltas = hash_values[1:] - hash_values[:-1]
    params = jnp.zeros((48,), jnp.float32)
    params = (params.at[0:nl - 1].set(t)
                    .at[16:16 + nl - 1].set(deltas)
                    .at[32].set(hash_values[0]))
    return _make_map(n, 16384, nl - 1)(x, params)


# speedup vs baseline: 6.0089x; 1.3142x over previous
import functools

import jax
import jax.numpy as jnp
from jax import lax
from jax.experimental import pallas as pl
from jax.experimental.pallas import tpu as pltpu
from jax.experimental.pallas import tpu_sc as plsc

_NW = 32
_L = 16


def _wid():
    return lax.axis_index("s") * 2 + lax.axis_index("c")


@functools.lru_cache(maxsize=None)
def _make_reduce(n, chunk):
    per_w = n // _NW
    nch = per_w // chunk
    mesh = plsc.VectorSubcoreMesh(core_axis_name="c", subcore_axis_name="s")

    @functools.partial(
        pl.kernel,
        mesh=mesh,
        out_type=jax.ShapeDtypeStruct((2, _NW, _L), jnp.float32),
        scratch_types=[
            pltpu.VMEM((chunk,), jnp.float32),
            pltpu.VMEM((chunk,), jnp.float32),
            pltpu.VMEM((_L,), jnp.float32),
            pltpu.VMEM((_L,), jnp.float32),
            pltpu.SemaphoreType.DMA,
            pltpu.SemaphoreType.DMA,
        ],
    )
    def reduce_k(x_hbm, out_hbm, buf0, buf1, sv, qv, sem0, sem1):
        wid = _wid()
        base = wid * per_w
        bufs = (buf0, buf1)
        sems = (sem0, sem1)
        copies = [None, None]
        copies[0] = pltpu.async_copy(x_hbm.at[pl.ds(base, chunk)], buf0, sem0)
        acc_s = jnp.zeros((_L,), jnp.float32)
        acc_q = jnp.zeros((_L,), jnp.float32)
        for ch in range(nch):
            b = ch % 2
            if ch + 1 < nch:
                nb = (ch + 1) % 2
                copies[nb] = pltpu.async_copy(
                    x_hbm.at[pl.ds(base + (ch + 1) * chunk, chunk)],
                    bufs[nb], sems[nb])
            copies[b].wait()
            buf = bufs[b]
            U = 8
            zz = tuple(jnp.zeros((_L,), jnp.float32) for _ in range(U))

            def body(j, carry):
                ss, qq = carry
                ns, nq = [], []
                for u in range(U):
                    v = buf[pl.ds(j * (U * _L) + u * _L, _L)]
                    ns.append(ss[u] + v)
                    nq.append(qq[u] + v * v)
                return tuple(ns), tuple(nq)

            ss, qq = lax.fori_loop(0, chunk // (U * _L), body, (zz, zz))
            for u in range(U):
                acc_s = acc_s + ss[u]
                acc_q = acc_q + qq[u]
        sv[...] = acc_s
        qv[...] = acc_q
        pltpu.sync_copy(sv, out_hbm.at[0, wid])
        pltpu.sync_copy(qv, out_hbm.at[1, wid])

    return reduce_k


_T = 256


@functools.lru_cache(maxsize=None)
def _make_map(n, chunk):
    per_w = n // _NW
    nch = per_w // chunk
    mesh = plsc.VectorSubcoreMesh(core_axis_name="c", subcore_axis_name="s")

    @functools.partial(
        pl.kernel,
        mesh=mesh,
        compiler_params=pltpu.CompilerParams(needs_layout_passes=False),
        out_type=jax.ShapeDtypeStruct((n,), jnp.float32),
        scratch_types=[
            pltpu.VMEM((chunk,), jnp.float32),
            pltpu.VMEM((chunk,), jnp.float32),
            pltpu.VMEM((chunk,), jnp.float32),
            pltpu.VMEM((chunk,), jnp.float32),
            pltpu.VMEM((_L,), jnp.float32),
            pltpu.VMEM((_T,), jnp.float32),
            pltpu.VMEM((2 * _T,), jnp.float32),
            pltpu.SemaphoreType.DMA,
            pltpu.SemaphoreType.DMA,
            pltpu.SemaphoreType.DMA,
            pltpu.SemaphoreType.DMA,
        ],
    )
    def map_k(x_hbm, par_hbm, out_hbm, ib0, ib1, ob0, ob1, pv, tt, tv,
              si0, si1, so0, so1):
        wid = _wid()
        base = wid * per_w
        pltpu.sync_copy(par_hbm.at[pl.ds(0, _L)], pv)
        pltpu.sync_copy(par_hbm.at[pl.ds(_L, _T)], tt)
        pltpu.sync_copy(par_hbm.at[pl.ds(_L + _T, 2 * _T)], tv)
        svec = pv[pl.ds(0, _L)]
        scale = svec[0]
        bias = svec[1]
        fmax_ = jnp.float32(_T - 1)
        ibufs = (ib0, ib1)
        obufs = (ob0, ob1)
        isems = (si0, si1)
        osems = (so0, so1)
        in_copies = [None, None]
        out_copies = [None, None]
        in_copies[0] = pltpu.async_copy(
            x_hbm.at[pl.ds(base, chunk)], ib0, si0)
        for ch in range(nch):
            b = ch % 2
            if ch + 1 < nch:
                nb = (ch + 1) % 2
                in_copies[nb] = pltpu.async_copy(
                    x_hbm.at[pl.ds(base + (ch + 1) * chunk, chunk)],
                    ibufs[nb], isems[nb])
            in_copies[b].wait()
            if out_copies[b] is not None:
                out_copies[b].wait()
            ib = ibufs[b]
            ob = obufs[b]

            @plsc.parallel_loop(0, chunk, _L, unroll=8)
            def _(i):
                v = ib[pl.ds(i, _L)]
                q = v * scale + bias
                q = jnp.minimum(jnp.maximum(q, jnp.float32(0.0)), fmax_)
                idx = q.astype(jnp.int32)
                t = plsc.load_gather(tt, [idx])
                one = jnp.full((_L,), 1, jnp.int32)
                zero = jnp.full((_L,), 0, jnp.int32)
                idx2 = idx + idx + jnp.where(v > t, one, zero)
                ob[pl.ds(i, _L)] = plsc.load_gather(tv, [idx2])

            out_copies[b] = pltpu.async_copy(
                obufs[b], out_hbm.at[pl.ds(base + ch * chunk, chunk)],
                osems[b])
        for oc in out_copies:
            if oc is not None:
                oc.wait()

    return map_k


def kernel(x, cutoffs, hash_values):
    n = x.shape[0]
    nl = hash_values.shape[0]
    x = x.astype(jnp.float32)
    part = _make_reduce(n, 16384)(x)
    s = jnp.sum(part[0])
    q = jnp.sum(part[1])
    mean = s / n
    var = (q - s * s / n) / (n - 1)
    denom = jnp.sqrt(var) + 1e-6
    t = cutoffs[1:-1] * denom + mean
    scale = jnp.float32(_T) / (16.0 * denom)
    bias = -(mean - 8.0 * denom) * scale
    fq = jnp.clip(t * scale + bias, 0.0, _T - 1)
    ci = fq.astype(jnp.int32)
    tcell = jnp.full((_T,), jnp.inf, jnp.float32).at[ci].set(t)
    cnt = jnp.zeros((_T,), jnp.int32).at[ci].add(1)
    nbelow = jnp.cumsum(cnt) - cnt
    val_lo = hash_values[jnp.clip(nbelow, 0, nl - 1)]
    val_hi = hash_values[jnp.clip(nbelow + 1, 0, nl - 1)]
    tbl_v = jnp.stack([val_lo, val_hi], axis=1).reshape(2 * _T)
    head = jnp.zeros((_L,), jnp.float32).at[0].set(scale).at[1].set(bias)
    params = jnp.concatenate([head, tcell, tbl_v])
    return _make_map(n, 16384)(x, params)


# speedup vs baseline: 6.1061x; 1.0162x over previous
import functools

import jax
import jax.numpy as jnp
from jax import lax
from jax.experimental import pallas as pl
from jax.experimental.pallas import tpu as pltpu
from jax.experimental.pallas import tpu_sc as plsc

_NW = 32
_L = 16


def _wid():
    return lax.axis_index("s") * 2 + lax.axis_index("c")


@functools.lru_cache(maxsize=None)
def _make_reduce(n, chunk):
    per_w = n // _NW
    nch = per_w // chunk
    mesh = plsc.VectorSubcoreMesh(core_axis_name="c", subcore_axis_name="s")

    @functools.partial(
        pl.kernel,
        mesh=mesh,
        out_type=jax.ShapeDtypeStruct((2, _NW, _L), jnp.float32),
        scratch_types=[
            pltpu.VMEM((chunk,), jnp.float32),
            pltpu.VMEM((chunk,), jnp.float32),
            pltpu.VMEM((_L,), jnp.float32),
            pltpu.VMEM((_L,), jnp.float32),
            pltpu.SemaphoreType.DMA,
            pltpu.SemaphoreType.DMA,
        ],
    )
    def reduce_k(x_hbm, out_hbm, buf0, buf1, sv, qv, sem0, sem1):
        wid = _wid()
        base = wid * per_w
        bufs = (buf0, buf1)
        sems = (sem0, sem1)
        copies = [None, None]
        copies[0] = pltpu.async_copy(x_hbm.at[pl.ds(base, chunk)], buf0, sem0)
        acc_s = jnp.zeros((_L,), jnp.float32)
        acc_q = jnp.zeros((_L,), jnp.float32)
        for ch in range(nch):
            b = ch % 2
            if ch + 1 < nch:
                nb = (ch + 1) % 2
                copies[nb] = pltpu.async_copy(
                    x_hbm.at[pl.ds(base + (ch + 1) * chunk, chunk)],
                    bufs[nb], sems[nb])
            copies[b].wait()
            buf = bufs[b]
            U = 8
            zz = tuple(jnp.zeros((_L,), jnp.float32) for _ in range(U))

            def body(j, carry):
                ss, qq = carry
                ns, nq = [], []
                for u in range(U):
                    v = buf[pl.ds(j * (U * _L) + u * _L, _L)]
                    ns.append(ss[u] + v)
                    nq.append(qq[u] + v * v)
                return tuple(ns), tuple(nq)

            ss, qq = lax.fori_loop(0, chunk // (U * _L), body, (zz, zz))
            for u in range(U):
                acc_s = acc_s + ss[u]
                acc_q = acc_q + qq[u]
        sv[...] = acc_s
        qv[...] = acc_q
        pltpu.sync_copy(sv, out_hbm.at[0, wid])
        pltpu.sync_copy(qv, out_hbm.at[1, wid])

    return reduce_k


_T = 256


@functools.lru_cache(maxsize=None)
def _make_map(n, chunk, nl):
    per_w = n // _NW
    nch = per_w // chunk
    nt = nl - 1
    mesh = plsc.VectorSubcoreMesh(core_axis_name="c", subcore_axis_name="s")

    @functools.partial(
        pl.kernel,
        mesh=mesh,
        compiler_params=pltpu.CompilerParams(needs_layout_passes=False),
        out_type=jax.ShapeDtypeStruct((n,), jnp.float32),
        scratch_types=[
            pltpu.VMEM((chunk,), jnp.float32),
            pltpu.VMEM((chunk,), jnp.float32),
            pltpu.VMEM((chunk,), jnp.float32),
            pltpu.VMEM((chunk,), jnp.float32),
            pltpu.VMEM((chunk,), jnp.float32),
            pltpu.VMEM((chunk,), jnp.float32),
            pltpu.VMEM((2, _NW, _L), jnp.float32),
            pltpu.VMEM((_L,), jnp.float32),
            pltpu.VMEM((_L,), jnp.float32),
            pltpu.VMEM((_T,), jnp.int32),
            pltpu.VMEM((_T,), jnp.float32),
            pltpu.VMEM((2 * _T,), jnp.float32),
            pltpu.SemaphoreType.DMA,
            pltpu.SemaphoreType.DMA,
            pltpu.SemaphoreType.DMA,
            pltpu.SemaphoreType.DMA,
            pltpu.SemaphoreType.DMA,
            pltpu.SemaphoreType.DMA,
        ],
    )
    def map_k(x_hbm, part_hbm, cut_hbm, hash_hbm, out_hbm,
              ib0, ib1, ib2, ob0, ob1, ob2, pb, cv16, hv, cnt, tt, tv,
              si0, si1, si2, so0, so1, so2):
        wid = _wid()
        base = wid * per_w
        nbuf = 3
        ibufs = (ib0, ib1, ib2)
        obufs = (ob0, ob1, ob2)
        isems = (si0, si1, si2)
        osems = (so0, so1, so2)
        in_copies = [None] * nbuf
        out_copies = [None] * nbuf
        for p in range(min(nbuf - 1, nch)):
            in_copies[p] = pltpu.async_copy(
                x_hbm.at[pl.ds(base + p * chunk, chunk)], ibufs[p], isems[p])

        pltpu.sync_copy(part_hbm, pb)
        pltpu.sync_copy(cut_hbm, cv16)
        pltpu.sync_copy(hash_hbm, hv)

        acc_s = jnp.zeros((_L,), jnp.float32)
        acc_q = jnp.zeros((_L,), jnp.float32)
        for j in range(_NW):
            acc_s = acc_s + pb[0, j]
            acc_q = acc_q + pb[1, j]
        s_tot = jnp.full((_L,), jnp.sum(acc_s), jnp.float32)
        q_tot = jnp.full((_L,), jnp.sum(acc_q), jnp.float32)
        mean = s_tot * jnp.float32(1.0 / n)
        var = (q_tot - s_tot * mean) * jnp.float32(1.0 / (n - 1))
        y = lax.bitcast_convert_type(
            (lax.bitcast_convert_type(var, jnp.int32) >> 1)
            + jnp.full((_L,), 0x1FBD1DF5, jnp.int32), jnp.float32)
        for _ in range(3):
            y = jnp.float32(0.5) * (y + var / y)
        denom = y + jnp.float32(1e-6)

        svec = jnp.float32(_T / 16.0) / denom
        bvec = (jnp.float32(8.0) * denom - mean) * svec
        lanes = lax.iota(jnp.int32, _L)
        valid = lanes < nt
        tvals = cv16[...] * denom + mean
        fq = jnp.minimum(jnp.maximum(tvals * svec + bvec, jnp.float32(0.0)),
                         jnp.float32(_T - 1))
        ci = fq.astype(jnp.int32)
        zero_i = jnp.zeros((_L,), jnp.int32)
        inf_v = jnp.full((_L,), jnp.inf, jnp.float32)
        for k in range(_T // _L):
            cnt[pl.ds(k * _L, _L)] = zero_i
            tt[pl.ds(k * _L, _L)] = inf_v
        plsc.addupdate_scatter(cnt, [ci], jnp.full((_L,), 1, jnp.int32),
                               mask=valid)
        plsc.store_scatter(tt, [ci], tvals, mask=valid)
        run = jnp.zeros((_L,), jnp.int32)
        for k in range(_T // _L):
            cells = lanes + jnp.full((_L,), k * _L, jnp.int32)
            cv = cnt[pl.ds(k * _L, _L)]
            cum = plsc.cumsum(cv)
            nb = cum - cv + run
            lo_v = plsc.load_gather(hv, [nb])
            hi_v = plsc.load_gather(
                hv, [jnp.minimum(nb + 1, jnp.full((_L,), nl - 1, jnp.int32))])
            c2 = cells + cells
            plsc.store_scatter(tv, [c2], lo_v)
            plsc.store_scatter(tv, [c2 + 1], hi_v)
            run = run + jnp.full((_L,), cum[_L - 1], jnp.int32)

        fmax_ = jnp.float32(_T - 1)
        fzero = jnp.float32(0.0)
        one = jnp.full((_L,), 1, jnp.int32)

        for ch in range(nch):
            b = ch % nbuf
            if ch + nbuf - 1 < nch:
                nb_ = (ch + nbuf - 1) % nbuf
                in_copies[nb_] = pltpu.async_copy(
                    x_hbm.at[pl.ds(base + (ch + nbuf - 1) * chunk, chunk)],
                    ibufs[nb_], isems[nb_])
            in_copies[b].wait()
            if out_copies[b] is not None:
                out_copies[b].wait()
            ib = ibufs[b]
            ob = obufs[b]

            @plsc.parallel_loop(0, chunk, _L, unroll=16)
            def _(i):
                v = ib[pl.ds(i, _L)]
                q = v * svec + bvec
                q = jnp.minimum(jnp.maximum(q, fzero), fmax_)
                idx = q.astype(jnp.int32)
                t = plsc.load_gather(tt, [idx])
                idx2 = idx + idx + jnp.where(v > t, one, zero_i)
                ob[pl.ds(i, _L)] = plsc.load_gather(tv, [idx2])

            out_copies[b] = pltpu.async_copy(
                obufs[b], out_hbm.at[pl.ds(base + ch * chunk, chunk)],
                osems[b])
        for oc in out_copies:
            if oc is not None:
                oc.wait()

    return map_k


def kernel(x, cutoffs, hash_values):
    n = x.shape[0]
    nl = hash_values.shape[0]
    x = x.astype(jnp.float32)
    part = _make_reduce(n, 16384)(x)
    cut16 = jnp.zeros((_L,), jnp.float32).at[0:nl - 1].set(cutoffs[1:-1])
    hash16 = jnp.zeros((_L,), jnp.float32).at[0:nl].set(hash_values)
    return _make_map(n, 16384, nl)(x, part, cut16, hash16)


# speedup vs baseline: 6.1699x; 1.0104x over previous
import functools

import jax
import jax.numpy as jnp
from jax import lax
from jax.experimental import pallas as pl
from jax.experimental.pallas import tpu as pltpu
from jax.experimental.pallas import tpu_sc as plsc

_NW = 32
_L = 16


def _wid():
    return lax.axis_index("s") * 2 + lax.axis_index("c")


@functools.lru_cache(maxsize=None)
def _make_reduce(n, chunk):
    per_w = n // _NW
    nch = per_w // chunk
    mesh = plsc.VectorSubcoreMesh(core_axis_name="c", subcore_axis_name="s")

    @functools.partial(
        pl.kernel,
        mesh=mesh,
        out_type=jax.ShapeDtypeStruct((2, _NW, _L), jnp.float32),
        scratch_types=[
            pltpu.VMEM((chunk,), jnp.float32),
            pltpu.VMEM((chunk,), jnp.float32),
            pltpu.VMEM((_L,), jnp.float32),
            pltpu.VMEM((_L,), jnp.float32),
            pltpu.SemaphoreType.DMA,
            pltpu.SemaphoreType.DMA,
        ],
    )
    def reduce_k(x_hbm, out_hbm, buf0, buf1, sv, qv, sem0, sem1):
        wid = _wid()
        base = wid * per_w
        bufs = (buf0, buf1)
        sems = (sem0, sem1)
        copies = [None, None]
        copies[0] = pltpu.async_copy(x_hbm.at[pl.ds(base, chunk)], buf0, sem0)
        acc_s = jnp.zeros((_L,), jnp.float32)
        acc_q = jnp.zeros((_L,), jnp.float32)
        for ch in range(nch):
            b = ch % 2
            if ch + 1 < nch:
                nb = (ch + 1) % 2
                copies[nb] = pltpu.async_copy(
                    x_hbm.at[pl.ds(base + (ch + 1) * chunk, chunk)],
                    bufs[nb], sems[nb])
            copies[b].wait()
            buf = bufs[b]
            U = 8
            zz = tuple(jnp.zeros((_L,), jnp.float32) for _ in range(U))

            def body(j, carry):
                ss, qq = carry
                ns, nq = [], []
                for u in range(U):
                    v = buf[pl.ds(j * (U * _L) + u * _L, _L)]
                    ns.append(ss[u] + v)
                    nq.append(qq[u] + v * v)
                return tuple(ns), tuple(nq)

            ss, qq = lax.fori_loop(0, chunk // (U * _L), body, (zz, zz))
            for u in range(U):
                acc_s = acc_s + ss[u]
                acc_q = acc_q + qq[u]
        sv[...] = acc_s
        qv[...] = acc_q
        pltpu.sync_copy(sv, out_hbm.at[0, wid])
        pltpu.sync_copy(qv, out_hbm.at[1, wid])

    return reduce_k


_T = 256


@functools.lru_cache(maxsize=None)
def _make_map(n, chunk, nl):
    per_w = n // _NW
    nch = per_w // chunk
    nt = nl - 1
    mesh = plsc.VectorSubcoreMesh(core_axis_name="c", subcore_axis_name="s")

    @functools.partial(
        pl.kernel,
        mesh=mesh,
        compiler_params=pltpu.CompilerParams(needs_layout_passes=False),
        out_type=jax.ShapeDtypeStruct((n,), jnp.float32),
        scratch_types=[
            pltpu.VMEM((chunk,), jnp.float32),
            pltpu.VMEM((chunk,), jnp.float32),
            pltpu.VMEM((chunk,), jnp.float32),
            pltpu.VMEM((chunk,), jnp.float32),
            pltpu.VMEM((chunk,), jnp.float32),
            pltpu.VMEM((chunk,), jnp.float32),
            pltpu.VMEM((2, _NW, _L), jnp.float32),
            pltpu.VMEM((_L,), jnp.float32),
            pltpu.VMEM((_L,), jnp.float32),
            pltpu.VMEM((_T,), jnp.int32),
            pltpu.VMEM((_T,), jnp.float32),
            pltpu.VMEM((2 * _T,), jnp.float32),
            pltpu.SemaphoreType.DMA,
            pltpu.SemaphoreType.DMA,
            pltpu.SemaphoreType.DMA,
            pltpu.SemaphoreType.DMA,
            pltpu.SemaphoreType.DMA,
            pltpu.SemaphoreType.DMA,
        ],
    )
    def map_k(x_hbm, part_hbm, cut_hbm, hash_hbm, out_hbm,
              ib0, ib1, ib2, ob0, ob1, ob2, pb, cv16, hv, cnt, tt, tv,
              si0, si1, si2, so0, so1, so2):
        wid = _wid()
        base = wid * per_w
        nbuf = 3
        ibufs = (ib0, ib1, ib2)
        obufs = (ob0, ob1, ob2)
        isems = (si0, si1, si2)
        osems = (so0, so1, so2)
        in_copies = [None] * nbuf
        out_copies = [None] * nbuf
        for p in range(min(nbuf - 1, nch)):
            in_copies[p] = pltpu.async_copy(
                x_hbm.at[pl.ds(base + p * chunk, chunk)], ibufs[p], isems[p])

        pltpu.sync_copy(part_hbm, pb)
        pltpu.sync_copy(cut_hbm, cv16)
        pltpu.sync_copy(hash_hbm, hv)

        acc_s = jnp.zeros((_L,), jnp.float32)
        acc_q = jnp.zeros((_L,), jnp.float32)
        for j in range(_NW):
            acc_s = acc_s + pb[0, j]
            acc_q = acc_q + pb[1, j]
        s_tot = jnp.full((_L,), jnp.sum(acc_s), jnp.float32)
        q_tot = jnp.full((_L,), jnp.sum(acc_q), jnp.float32)
        mean = s_tot * jnp.float32(1.0 / n)
        var = (q_tot - s_tot * mean) * jnp.float32(1.0 / (n - 1))
        y = lax.bitcast_convert_type(
            (lax.bitcast_convert_type(var, jnp.int32) >> 1)
            + jnp.full((_L,), 0x1FBD1DF5, jnp.int32), jnp.float32)
        for _ in range(3):
            y = jnp.float32(0.5) * (y + var / y)
        denom = y + jnp.float32(1e-6)

        svec = jnp.float32(_T / 16.0) / denom
        bvec = (jnp.float32(8.0) * denom - mean) * svec
        lanes = lax.iota(jnp.int32, _L)
        valid = lanes < nt
        tvals = cv16[...] * denom + mean
        fq = jnp.minimum(jnp.maximum(tvals * svec + bvec, jnp.float32(0.0)),
                         jnp.float32(_T - 1))
        ci = fq.astype(jnp.int32)
        zero_i = jnp.zeros((_L,), jnp.int32)
        inf_v = jnp.full((_L,), jnp.inf, jnp.float32)
        for k in range(_T // _L):
            cnt[pl.ds(k * _L, _L)] = zero_i
            tt[pl.ds(k * _L, _L)] = inf_v
        plsc.addupdate_scatter(cnt, [ci], jnp.full((_L,), 1, jnp.int32),
                               mask=valid)
        plsc.store_scatter(tt, [ci], tvals, mask=valid)
        run = jnp.zeros((_L,), jnp.int32)
        for k in range(_T // _L):
            cells = lanes + jnp.full((_L,), k * _L, jnp.int32)
            cv = cnt[pl.ds(k * _L, _L)]
            cum = plsc.cumsum(cv)
            nb = cum - cv + run
            lo_v = plsc.load_gather(hv, [nb])
            hi_v = plsc.load_gather(
                hv, [jnp.minimum(nb + 1, jnp.full((_L,), nl - 1, jnp.int32))])
            c2 = cells + cells
            plsc.store_scatter(tv, [c2], lo_v)
            plsc.store_scatter(tv, [c2 + 1], hi_v)
            run = run + jnp.full((_L,), cum[_L - 1], jnp.int32)

        fmax_ = jnp.float32(_T - 1)
        fzero = jnp.float32(0.0)
        one = jnp.full((_L,), 1, jnp.int32)

        for ch in range(nch):
            b = ch % nbuf
            if ch + nbuf - 1 < nch:
                nb_ = (ch + nbuf - 1) % nbuf
                in_copies[nb_] = pltpu.async_copy(
                    x_hbm.at[pl.ds(base + (ch + nbuf - 1) * chunk, chunk)],
                    ibufs[nb_], isems[nb_])
            in_copies[b].wait()
            if out_copies[b] is not None:
                out_copies[b].wait()
            ib = ibufs[b]
            ob = obufs[b]

            @plsc.parallel_loop(0, chunk, _L, unroll=8)
            def _(i):
                v = ib[pl.ds(i, _L)]
                q = v * svec + bvec
                q = jnp.minimum(jnp.maximum(q, fzero), fmax_)
                idx = q.astype(jnp.int32)
                t = plsc.load_gather(tt, [idx])
                idx2 = idx + idx + jnp.where(v > t, one, zero_i)
                ob[pl.ds(i, _L)] = plsc.load_gather(tv, [idx2])

            out_copies[b] = pltpu.async_copy(
                obufs[b], out_hbm.at[pl.ds(base + ch * chunk, chunk)],
                osems[b])
        for oc in out_copies:
            if oc is not None:
                oc.wait()

    return map_k


def kernel(x, cutoffs, hash_values):
    n = x.shape[0]
    nl = hash_values.shape[0]
    x = x.astype(jnp.float32)
    part = _make_reduce(n, 16384)(x)
    cut16 = jnp.zeros((_L,), jnp.float32).at[0:nl - 1].set(cutoffs[1:-1])
    hash16 = jnp.zeros((_L,), jnp.float32).at[0:nl].set(hash_values)
    return _make_map(n, 16384, nl)(x, part, cut16, hash16)
